# Initial kernel scaffold; baseline (speedup 1.0000x reference)
import jax
import jax.numpy as jnp
from jax.experimental import pallas as pl


def kernel(x, edge_index, W1, b1, W2, b2):
    raise NotImplementedError("write your pallas kernel here")


# speedup vs baseline: 67.2836x; 67.2836x over previous
import jax
import jax.numpy as jnp
from jax import lax
from jax.experimental import pallas as pl
from jax.experimental.pallas import tpu as pltpu
from jax.experimental.pallas import tpu_sc as plsc

N = 10000
E = 320000
D = 128
H = 10
HP = 16
FH = 5
L = 16
NC, NS = 2, 16
NW = NC * NS
CH = 10000

_mesh = plsc.VectorSubcoreMesh(
    core_axis_name="c", subcore_axis_name="s", num_cores=NC, num_subcores=NS
)
_sc_params = pltpu.CompilerParams(needs_layout_passes=False)


def _deg_body(dst_hbm, zeros_hbm, out_hbm, dst_v, acc_v):
    wid = lax.axis_index("c") * NS + lax.axis_index("s")
    epw = E // NW
    pltpu.sync_copy(zeros_hbm, acc_v)
    pltpu.sync_copy(dst_hbm.at[pl.ds(wid * epw, epw)], dst_v)
    ones = jnp.ones((L,), jnp.float32)

    def body(i, carry):
        dv = dst_v[pl.ds(i * L, L)]
        plsc.addupdate_scatter(acc_v, [dv], ones)
        return carry

    lax.fori_loop(0, epw // L, body, 0)
    pltpu.sync_copy(acc_v, out_hbm.at[pl.ds(wid * N, N)])


_deg_call = pl.kernel(
    _deg_body,
    out_type=jax.ShapeDtypeStruct((NW * N,), jnp.float32),
    mesh=_mesh,
    compiler_params=_sc_params,
    scratch_types=[
        pltpu.VMEM((E // NW,), jnp.int32),
        pltpu.VMEM((N,), jnp.float32),
    ],
)


def _edge1_body(g1t_hbm, src_hbm, dst_hbm, zeros_hbm, out_hbm,
                g1h_v, acc_v, src_v, dst_v):
    c = lax.axis_index("c")
    s = lax.axis_index("s")
    epw = E // NS
    base = s * epw
    pltpu.sync_copy(g1t_hbm.at[pl.ds(c * (FH * N), FH * N)], g1h_v)
    pltpu.sync_copy(zeros_hbm, acc_v)
    for ch in range(epw // CH):
        pltpu.sync_copy(src_hbm.at[pl.ds(base + ch * CH, CH)], src_v)
        pltpu.sync_copy(dst_hbm.at[pl.ds(base + ch * CH, CH)], dst_v)

        def body(i, carry):
            sv = src_v[pl.ds(i * L, L)]
            dv = dst_v[pl.ds(i * L, L)]
            for j in range(FH):
                off = jnp.full((L,), j * N, jnp.int32)
                vals = plsc.load_gather(g1h_v, [sv + off])
                plsc.addupdate_scatter(acc_v, [dv + off], vals)
            return carry

        lax.fori_loop(0, CH // L, body, 0)
    pltpu.sync_copy(acc_v, out_hbm.at[pl.ds((s * H + c * FH) * N, FH * N)])


_edge1_call = pl.kernel(
    _edge1_body,
    out_type=jax.ShapeDtypeStruct((NS * H * N,), jnp.float32),
    mesh=_mesh,
    compiler_params=_sc_params,
    scratch_types=[
        pltpu.VMEM((FH * N,), jnp.float32),
        pltpu.VMEM((FH * N,), jnp.float32),
        pltpu.VMEM((CH,), jnp.int32),
        pltpu.VMEM((CH,), jnp.int32),
    ],
)


def _edge2_body(g2_hbm, src_hbm, dst_hbm, zeros_hbm, out_hbm,
                g2_v, acc_v, src_v, dst_v):
    wid = lax.axis_index("c") * NS + lax.axis_index("s")
    epw = E // NW
    base = wid * epw
    pltpu.sync_copy(g2_hbm, g2_v)
    pltpu.sync_copy(zeros_hbm, acc_v)
    pltpu.sync_copy(src_hbm.at[pl.ds(base, epw)], src_v)
    pltpu.sync_copy(dst_hbm.at[pl.ds(base, epw)], dst_v)

    def body(i, carry):
        sv = src_v[pl.ds(i * L, L)]
        dv = dst_v[pl.ds(i * L, L)]
        vals = plsc.load_gather(g2_v, [sv])
        plsc.addupdate_scatter(acc_v, [dv], vals)
        return carry

    lax.fori_loop(0, epw // L, body, 0)
    pltpu.sync_copy(acc_v, out_hbm.at[pl.ds(wid * N, N)])


_edge2_call = pl.kernel(
    _edge2_body,
    out_type=jax.ShapeDtypeStruct((NW * N,), jnp.float32),
    mesh=_mesh,
    compiler_params=_sc_params,
    scratch_types=[
        pltpu.VMEM((N,), jnp.float32),
        pltpu.VMEM((N,), jnp.float32),
        pltpu.VMEM((E // NW,), jnp.int32),
        pltpu.VMEM((E // NW,), jnp.int32),
    ],
)


def _mm1_body(x_ref, w1_ref, degp_ref, g1t_ref, h1t_ref, dinv_ref):
    deg = jnp.sum(degp_ref[...], axis=0, keepdims=True) + 1.0
    dinv = lax.rsqrt(deg)
    h1 = jnp.dot(x_ref[...], w1_ref[...],
                 preferred_element_type=jnp.float32)
    h1t = h1.T
    g1t_ref[...] = h1t * dinv
    h1t_ref[...] = h1t
    dinv_ref[...] = dinv


_mm1_call = pl.pallas_call(
    _mm1_body,
    out_shape=[
        jax.ShapeDtypeStruct((HP, N), jnp.float32),
        jax.ShapeDtypeStruct((HP, N), jnp.float32),
        jax.ShapeDtypeStruct((1, N), jnp.float32),
    ],
)


def _mm2_body(p1p_ref, h1t_ref, dinv_ref, b1_ref, w2t_ref, h2_ref, g2_ref):
    p1 = jnp.sum(p1p_ref[...], axis=0)
    dinv = dinv_ref[...]
    h1t = h1t_ref[...][:H, :]
    out1 = p1 * dinv + (dinv * dinv) * h1t + b1_ref[...]
    out1 = jnp.maximum(out1, 0.0)
    h2 = jnp.dot(w2t_ref[...], out1,
                 preferred_element_type=jnp.float32)
    h2_ref[...] = h2
    g2_ref[...] = h2 * dinv


_mm2_call = pl.pallas_call(
    _mm2_body,
    out_shape=[
        jax.ShapeDtypeStruct((1, N), jnp.float32),
        jax.ShapeDtypeStruct((1, N), jnp.float32),
    ],
)


def _fin_body(p2p_ref, h2_ref, dinv_ref, b2_ref, out_ref):
    dinv = dinv_ref[...]
    col = (dinv * jnp.sum(p2p_ref[...], axis=0, keepdims=True)
           + (dinv * dinv) * h2_ref[...] + b2_ref[...])
    e0 = (lax.broadcasted_iota(jnp.int32, (1, D), 1) == 0)
    out_ref[...] = lax.dot_general(
        col, e0.astype(jnp.float32), (((0,), (0,)), ((), ())),
        preferred_element_type=jnp.float32)


_fin_call = pl.pallas_call(
    _fin_body,
    out_shape=jax.ShapeDtypeStruct((N, D), jnp.float32),
)


def kernel(x, edge_index, W1, b1, W2, b2):
    ei = edge_index.astype(jnp.int32)
    src, dst = ei[0], ei[1]
    w1p = jnp.pad(W1.astype(jnp.float32), ((0, 0), (0, HP - H)))
    b1c = b1.astype(jnp.float32).reshape(H, 1)
    w2t = W2.astype(jnp.float32).reshape(1, H)
    b2c = b2.astype(jnp.float32).reshape(1, 1)
    z_n = jnp.zeros((N,), jnp.float32)
    z_fh = jnp.zeros((FH * N,), jnp.float32)

    degp = _deg_call(dst, z_n).reshape(NW, N)
    g1t, h1t, dinv = _mm1_call(x, w1p, degp)
    p1p = _edge1_call(g1t.reshape(HP * N)[: H * N], src, dst, z_fh)
    h2, g2 = _mm2_call(p1p.reshape(NS, H, N), h1t, dinv, b1c, w2t)
    p2p = _edge2_call(g2.reshape(N), src, dst, z_n)
    out = _fin_call(p2p.reshape(NW, N), h2, dinv, b2c)
    return out


# speedup vs baseline: 88.0374x; 1.3085x over previous
import jax
import jax.numpy as jnp
from jax import lax
from jax.experimental import pallas as pl
from jax.experimental.pallas import tpu as pltpu
from jax.experimental.pallas import tpu_sc as plsc

N = 10000
E = 320000
D = 128
H = 10
HP = 16
FH = 5
L = 16
NC, NS = 2, 16
NW = NC * NS
CH = 10000

_mesh = plsc.VectorSubcoreMesh(
    core_axis_name="c", subcore_axis_name="s", num_cores=NC, num_subcores=NS
)
_sc_params = pltpu.CompilerParams(needs_layout_passes=False)


def _deg_body(ei_hbm, zeros_hbm, out_hbm, dst_v, acc_v):
    wid = lax.axis_index("c") * NS + lax.axis_index("s")
    epw = E // NW
    pltpu.sync_copy(zeros_hbm, acc_v)
    pltpu.sync_copy(ei_hbm.at[pl.ds(E + wid * epw, epw)], dst_v)
    ones = jnp.ones((L,), jnp.float32)

    @plsc.parallel_loop(0, epw // L, unroll=8)
    def body(i):
        dv = dst_v[pl.ds(i * L, L)]
        plsc.addupdate_scatter(acc_v, [dv], ones)

    pltpu.sync_copy(acc_v, out_hbm.at[pl.ds(wid * N, N)])


_deg_call = pl.kernel(
    _deg_body,
    out_type=jax.ShapeDtypeStruct((NW * N,), jnp.float32),
    mesh=_mesh,
    compiler_params=_sc_params,
    scratch_types=[
        pltpu.VMEM((E // NW,), jnp.int32),
        pltpu.VMEM((N,), jnp.float32),
    ],
)


def _edge1_body(g1t_hbm, ei_hbm, zeros_hbm, out_hbm, *refs):
    g1_refs = refs[0:FH]
    acc_refs = refs[FH:2 * FH]
    src_v, dst_v = refs[2 * FH], refs[2 * FH + 1]
    c = lax.axis_index("c")
    s = lax.axis_index("s")
    epw = E // NS
    base = s * epw
    for j in range(FH):
        pltpu.sync_copy(g1t_hbm.at[pl.ds((c * FH + j) * N, N)], g1_refs[j])
        pltpu.sync_copy(zeros_hbm, acc_refs[j])
    for ch in range(epw // CH):
        pltpu.sync_copy(ei_hbm.at[pl.ds(base + ch * CH, CH)], src_v)
        pltpu.sync_copy(ei_hbm.at[pl.ds(E + base + ch * CH, CH)], dst_v)

        @plsc.parallel_loop(0, CH // L, unroll=8)
        def body(i):
            sv = src_v[pl.ds(i * L, L)]
            dv = dst_v[pl.ds(i * L, L)]
            for j in range(FH):
                vals = plsc.load_gather(g1_refs[j], [sv])
                plsc.addupdate_scatter(acc_refs[j], [dv], vals)

    for j in range(FH):
        pltpu.sync_copy(acc_refs[j],
                        out_hbm.at[pl.ds((s * H + c * FH + j) * N, N)])


_edge1_call = pl.kernel(
    _edge1_body,
    out_type=jax.ShapeDtypeStruct((NS * H * N,), jnp.float32),
    mesh=_mesh,
    compiler_params=_sc_params,
    scratch_types=(
        [pltpu.VMEM((N,), jnp.float32)] * (2 * FH)
        + [pltpu.VMEM((CH,), jnp.int32)] * 2
    ),
)


def _edge2_body(g2_hbm, ei_hbm, zeros_hbm, out_hbm,
                g2_v, acc_v, src_v, dst_v):
    wid = lax.axis_index("c") * NS + lax.axis_index("s")
    epw = E // NW
    base = wid * epw
    pltpu.sync_copy(g2_hbm, g2_v)
    pltpu.sync_copy(zeros_hbm, acc_v)
    pltpu.sync_copy(ei_hbm.at[pl.ds(base, epw)], src_v)
    pltpu.sync_copy(ei_hbm.at[pl.ds(E + base, epw)], dst_v)

    @plsc.parallel_loop(0, epw // L, unroll=8)
    def body(i):
        sv = src_v[pl.ds(i * L, L)]
        dv = dst_v[pl.ds(i * L, L)]
        vals = plsc.load_gather(g2_v, [sv])
        plsc.addupdate_scatter(acc_v, [dv], vals)

    pltpu.sync_copy(acc_v, out_hbm.at[pl.ds(wid * N, N)])


_edge2_call = pl.kernel(
    _edge2_body,
    out_type=jax.ShapeDtypeStruct((NW * N,), jnp.float32),
    mesh=_mesh,
    compiler_params=_sc_params,
    scratch_types=[
        pltpu.VMEM((N,), jnp.float32),
        pltpu.VMEM((N,), jnp.float32),
        pltpu.VMEM((E // NW,), jnp.int32),
        pltpu.VMEM((E // NW,), jnp.int32),
    ],
)


def _mm1_body(x_ref, w1_ref, degp_ref, g1t_ref, h1t_ref, dinv_ref):
    deg = jnp.sum(degp_ref[...], axis=0, keepdims=True) + 1.0
    dinv = lax.rsqrt(deg)
    h1 = jnp.dot(x_ref[...], w1_ref[...],
                 preferred_element_type=jnp.float32)
    h1t = h1.T
    g1t_ref[...] = h1t * dinv
    h1t_ref[...] = h1t
    dinv_ref[...] = dinv


_mm1_call = pl.pallas_call(
    _mm1_body,
    out_shape=[
        jax.ShapeDtypeStruct((HP, N), jnp.float32),
        jax.ShapeDtypeStruct((HP, N), jnp.float32),
        jax.ShapeDtypeStruct((1, N), jnp.float32),
    ],
)


def _mm2_body(p1p_ref, h1t_ref, dinv_ref, b1_ref, w2t_ref, h2_ref, g2_ref):
    p1 = jnp.sum(p1p_ref[...], axis=0)
    dinv = dinv_ref[...]
    h1t = h1t_ref[...][:H, :]
    out1 = p1 * dinv + (dinv * dinv) * h1t + b1_ref[...]
    out1 = jnp.maximum(out1, 0.0)
    h2 = jnp.dot(w2t_ref[...], out1,
                 preferred_element_type=jnp.float32)
    h2_ref[...] = h2
    g2_ref[...] = h2 * dinv


_mm2_call = pl.pallas_call(
    _mm2_body,
    out_shape=[
        jax.ShapeDtypeStruct((1, N), jnp.float32),
        jax.ShapeDtypeStruct((1, N), jnp.float32),
    ],
)


def _fin_body(p2p_ref, h2_ref, dinv_ref, b2_ref, out_ref):
    dinv = dinv_ref[...]
    col = (dinv * jnp.sum(p2p_ref[...], axis=0, keepdims=True)
           + (dinv * dinv) * h2_ref[...] + b2_ref[...])
    e0 = (lax.broadcasted_iota(jnp.int32, (1, D), 1) == 0)
    out_ref[...] = lax.dot_general(
        col, e0.astype(jnp.float32), (((0,), (0,)), ((), ())),
        preferred_element_type=jnp.float32)


_fin_call = pl.pallas_call(
    _fin_body,
    out_shape=jax.ShapeDtypeStruct((N, D), jnp.float32),
)


def kernel(x, edge_index, W1, b1, W2, b2):
    ei = edge_index.astype(jnp.int32).reshape(2 * E)
    w1p = jnp.pad(W1.astype(jnp.float32), ((0, 0), (0, HP - H)))
    b1c = b1.astype(jnp.float32).reshape(H, 1)
    w2t = W2.astype(jnp.float32).reshape(1, H)
    b2c = b2.astype(jnp.float32).reshape(1, 1)
    z_n = jnp.zeros((N,), jnp.float32)

    degp = _deg_call(ei, z_n).reshape(NW, N)
    g1t, h1t, dinv = _mm1_call(x, w1p, degp)
    p1p = _edge1_call(g1t.reshape(HP * N)[: H * N], ei, z_n)
    h2, g2 = _mm2_call(p1p.reshape(NS, H, N), h1t, dinv, b1c, w2t)
    p2p = _edge2_call(g2.reshape(N), ei, z_n)
    out = _fin_call(p2p.reshape(NW, N), h2, dinv, b2c)
    return out


# speedup vs baseline: 112.5338x; 1.2782x over previous
import jax
import jax.numpy as jnp
from jax import lax
from jax.experimental import pallas as pl
from jax.experimental.pallas import tpu as pltpu
from jax.experimental.pallas import tpu_sc as plsc

N = 10000
E = 320000
D = 128
H = 10
NP = 10240
HP = 16
FH = 5
L = 16
NC, NS = 2, 16
NW = NC * NS
CH = 10000

_mesh = plsc.VectorSubcoreMesh(
    core_axis_name="c", subcore_axis_name="s", num_cores=NC, num_subcores=NS
)
_sc_params = pltpu.CompilerParams(needs_layout_passes=False)


def _deg_body(ei_hbm, zeros_hbm, out_hbm, dst_v, acc_v):
    wid = lax.axis_index("c") * NS + lax.axis_index("s")
    epw = E // NW
    pltpu.sync_copy(zeros_hbm, acc_v)
    pltpu.sync_copy(ei_hbm.at[pl.ds(E + wid * epw, epw)], dst_v)
    ones = jnp.ones((L,), jnp.float32)

    @plsc.parallel_loop(0, epw // L, unroll=8)
    def body(i):
        dv = dst_v[pl.ds(i * L, L)]
        plsc.addupdate_scatter(acc_v, [dv], ones)

    pltpu.sync_copy(acc_v, out_hbm.at[pl.ds(wid * NP, NP)])


_deg_call = pl.kernel(
    _deg_body,
    out_type=jax.ShapeDtypeStruct((NW * NP,), jnp.float32),
    mesh=_mesh,
    compiler_params=_sc_params,
    scratch_types=[
        pltpu.VMEM((E // NW,), jnp.int32),
        pltpu.VMEM((NP,), jnp.float32),
    ],
)


def _edge1_body(g1t_hbm, ei_hbm, zeros_hbm, out_hbm, *refs):
    g1_refs = refs[0:FH]
    acc_refs = refs[FH:2 * FH]
    src_v, dst_v = refs[2 * FH], refs[2 * FH + 1]
    c = lax.axis_index("c")
    s = lax.axis_index("s")
    epw = E // NS
    base = s * epw
    for j in range(FH):
        pltpu.sync_copy(g1t_hbm.at[pl.ds((c * FH + j) * NP, NP)], g1_refs[j])
        pltpu.sync_copy(zeros_hbm, acc_refs[j])
    for ch in range(epw // CH):
        pltpu.sync_copy(ei_hbm.at[pl.ds(base + ch * CH, CH)], src_v)
        pltpu.sync_copy(ei_hbm.at[pl.ds(E + base + ch * CH, CH)], dst_v)

        @plsc.parallel_loop(0, CH // L, unroll=8)
        def body(i):
            sv = src_v[pl.ds(i * L, L)]
            dv = dst_v[pl.ds(i * L, L)]
            for j in range(FH):
                vals = plsc.load_gather(g1_refs[j], [sv])
                plsc.addupdate_scatter(acc_refs[j], [dv], vals)

    for j in range(FH):
        pltpu.sync_copy(acc_refs[j],
                        out_hbm.at[pl.ds((s * H + c * FH + j) * NP, NP)])


_edge1_call = pl.kernel(
    _edge1_body,
    out_type=jax.ShapeDtypeStruct((NS * H * NP,), jnp.float32),
    mesh=_mesh,
    compiler_params=_sc_params,
    scratch_types=(
        [pltpu.VMEM((NP,), jnp.float32)] * (2 * FH)
        + [pltpu.VMEM((CH,), jnp.int32)] * 2
    ),
)


def _edge2_body(g2_hbm, ei_hbm, zeros_hbm, out_hbm,
                g2_v, acc_v, src_v, dst_v):
    wid = lax.axis_index("c") * NS + lax.axis_index("s")
    epw = E // NW
    base = wid * epw
    pltpu.sync_copy(g2_hbm, g2_v)
    pltpu.sync_copy(zeros_hbm, acc_v)
    pltpu.sync_copy(ei_hbm.at[pl.ds(base, epw)], src_v)
    pltpu.sync_copy(ei_hbm.at[pl.ds(E + base, epw)], dst_v)

    @plsc.parallel_loop(0, epw // L, unroll=8)
    def body(i):
        sv = src_v[pl.ds(i * L, L)]
        dv = dst_v[pl.ds(i * L, L)]
        vals = plsc.load_gather(g2_v, [sv])
        plsc.addupdate_scatter(acc_v, [dv], vals)

    pltpu.sync_copy(acc_v, out_hbm.at[pl.ds(wid * NP, NP)])


_edge2_call = pl.kernel(
    _edge2_body,
    out_type=jax.ShapeDtypeStruct((NW * NP,), jnp.float32),
    mesh=_mesh,
    compiler_params=_sc_params,
    scratch_types=[
        pltpu.VMEM((NP,), jnp.float32),
        pltpu.VMEM((NP,), jnp.float32),
        pltpu.VMEM((E // NW,), jnp.int32),
        pltpu.VMEM((E // NW,), jnp.int32),
    ],
)


def _mm1_body(x_ref, w1_ref, degp_ref, g1t_ref, h1t_ref, dinv_ref):
    degp = degp_ref[...].reshape(NW, NP)
    deg = jnp.sum(degp, axis=0, keepdims=True) + 1.0
    dinv = lax.rsqrt(deg)
    h1 = jnp.dot(x_ref[...], w1_ref[...],
                 preferred_element_type=jnp.float32)
    h1t = jnp.concatenate(
        [h1.T, jnp.zeros((HP, NP - N), jnp.float32)], axis=1)
    g1t_ref[...] = (h1t * dinv).reshape(HP * NP)
    h1t_ref[...] = h1t
    dinv_ref[...] = dinv


_mm1_call = pl.pallas_call(
    _mm1_body,
    out_shape=[
        jax.ShapeDtypeStruct((HP * NP,), jnp.float32),
        jax.ShapeDtypeStruct((HP, NP), jnp.float32),
        jax.ShapeDtypeStruct((1, NP), jnp.float32),
    ],
)


def _mm2_body(p1p_ref, h1t_ref, dinv_ref, b1_ref, w2t_ref, h2_ref, g2_ref):
    p1p = p1p_ref[...].reshape(NS, H, NP)
    p1 = jnp.sum(p1p, axis=0)
    dinv = dinv_ref[...]
    h1t = h1t_ref[...][:H, :]
    out1 = p1 * dinv + (dinv * dinv) * h1t + b1_ref[...]
    out1 = jnp.maximum(out1, 0.0)
    h2 = jnp.dot(w2t_ref[...], out1,
                 preferred_element_type=jnp.float32)
    h2_ref[...] = h2
    g2_ref[...] = (h2 * dinv).reshape(NP)


_mm2_call = pl.pallas_call(
    _mm2_body,
    out_shape=[
        jax.ShapeDtypeStruct((1, NP), jnp.float32),
        jax.ShapeDtypeStruct((NP,), jnp.float32),
    ],
)


def _fin_body(p2p_ref, h2_ref, dinv_ref, b2_ref, out_ref):
    p2p = p2p_ref[...].reshape(NW, NP)
    dinv = dinv_ref[...]
    col = (dinv * jnp.sum(p2p, axis=0, keepdims=True)
           + (dinv * dinv) * h2_ref[...] + b2_ref[...])
    e0 = (lax.broadcasted_iota(jnp.int32, (1, D), 1) == 0)
    out_ref[...] = lax.dot_general(
        col[:, :N], e0.astype(jnp.float32), (((0,), (0,)), ((), ())),
        preferred_element_type=jnp.float32)


_fin_call = pl.pallas_call(
    _fin_body,
    out_shape=jax.ShapeDtypeStruct((N, D), jnp.float32),
)


def kernel(x, edge_index, W1, b1, W2, b2):
    ei = edge_index.astype(jnp.int32).reshape(2 * E)
    w1p = jnp.pad(W1.astype(jnp.float32), ((0, 0), (0, HP - H)))
    b1c = b1.astype(jnp.float32).reshape(H, 1)
    w2t = W2.astype(jnp.float32).reshape(1, H)
    b2c = b2.astype(jnp.float32).reshape(1, 1)
    z_np = jnp.zeros((NP,), jnp.float32)

    degp = _deg_call(ei, z_np)
    g1t, h1t, dinv = _mm1_call(x, w1p, degp)
    p1p = _edge1_call(g1t, ei, z_np)
    h2, g2 = _mm2_call(p1p, h1t, dinv, b1c, w2t)
    p2p = _edge2_call(g2, ei, z_np)
    out = _fin_call(p2p, h2, dinv, b2c)
    return out


# speedup vs baseline: 114.4645x; 1.0172x over previous
import jax
import jax.numpy as jnp
from jax import lax
from jax.experimental import pallas as pl
from jax.experimental.pallas import tpu as pltpu
from jax.experimental.pallas import tpu_sc as plsc

N = 10000
E = 320000
D = 128
H = 10
NP = 10240
HP = 16
FH = 5
L = 16
NC, NS = 2, 16
NW = NC * NS
CH = 10000

_mesh = plsc.VectorSubcoreMesh(
    core_axis_name="c", subcore_axis_name="s", num_cores=NC, num_subcores=NS
)
_sc_params = pltpu.CompilerParams(needs_layout_passes=False)


def _deg_body(ei_hbm, zeros_hbm, out_hbm, dst_v, acc_v):
    wid = lax.axis_index("c") * NS + lax.axis_index("s")
    epw = E // NW
    pltpu.sync_copy(zeros_hbm, acc_v)
    pltpu.sync_copy(ei_hbm.at[pl.ds(E + wid * epw, epw)], dst_v)
    ones = jnp.ones((L,), jnp.float32)

    @plsc.parallel_loop(0, epw // L, unroll=8)
    def body(i):
        dv = dst_v[pl.ds(i * L, L)]
        plsc.addupdate_scatter(acc_v, [dv], ones)

    pltpu.sync_copy(acc_v, out_hbm.at[pl.ds(wid * NP, NP)])


_deg_call = pl.kernel(
    _deg_body,
    out_type=jax.ShapeDtypeStruct((NW * NP,), jnp.float32),
    mesh=_mesh,
    compiler_params=_sc_params,
    scratch_types=[
        pltpu.VMEM((E // NW,), jnp.int32),
        pltpu.VMEM((NP,), jnp.float32),
    ],
)


def _edge1_body(g1t_hbm, ei_hbm, zeros_hbm, out_hbm, *refs):
    g1_refs = refs[0:FH]
    acc_refs = refs[FH:2 * FH]
    src_v, dst_v = refs[2 * FH], refs[2 * FH + 1]
    c = lax.axis_index("c")
    s = lax.axis_index("s")
    epw = E // NS
    base = s * epw
    for j in range(FH):
        pltpu.sync_copy(g1t_hbm.at[pl.ds((c * FH + j) * NP, NP)], g1_refs[j])
        pltpu.sync_copy(zeros_hbm, acc_refs[j])
    for ch in range(epw // CH):
        pltpu.sync_copy(ei_hbm.at[pl.ds(base + ch * CH, CH)], src_v)
        pltpu.sync_copy(ei_hbm.at[pl.ds(E + base + ch * CH, CH)], dst_v)

        @plsc.parallel_loop(0, CH // L, unroll=8)
        def body(i):
            sv = src_v[pl.ds(i * L, L)]
            dv = dst_v[pl.ds(i * L, L)]
            for j in range(FH):
                vals = plsc.load_gather(g1_refs[j], [sv])
                plsc.addupdate_scatter(acc_refs[j], [dv], vals)

    for j in range(FH):
        pltpu.sync_copy(acc_refs[j],
                        out_hbm.at[pl.ds((s * H + c * FH + j) * NP, NP)])


_edge1_call = pl.kernel(
    _edge1_body,
    out_type=jax.ShapeDtypeStruct((NS * H * NP,), jnp.float32),
    mesh=_mesh,
    compiler_params=_sc_params,
    scratch_types=(
        [pltpu.VMEM((NP,), jnp.float32)] * (2 * FH)
        + [pltpu.VMEM((CH,), jnp.int32)] * 2
    ),
)


def _edge2_body(g2_hbm, ei_hbm, zeros_hbm, out_hbm,
                g2_v, acc_v, src_v, dst_v):
    wid = lax.axis_index("c") * NS + lax.axis_index("s")
    epw = E // NW
    base = wid * epw
    pltpu.sync_copy(g2_hbm, g2_v)
    pltpu.sync_copy(zeros_hbm, acc_v)
    pltpu.sync_copy(ei_hbm.at[pl.ds(base, epw)], src_v)
    pltpu.sync_copy(ei_hbm.at[pl.ds(E + base, epw)], dst_v)

    @plsc.parallel_loop(0, epw // L, unroll=8)
    def body(i):
        sv = src_v[pl.ds(i * L, L)]
        dv = dst_v[pl.ds(i * L, L)]
        vals = plsc.load_gather(g2_v, [sv])
        plsc.addupdate_scatter(acc_v, [dv], vals)

    pltpu.sync_copy(acc_v, out_hbm.at[pl.ds(wid * NP, NP)])


_edge2_call = pl.kernel(
    _edge2_body,
    out_type=jax.ShapeDtypeStruct((NW * NP,), jnp.float32),
    mesh=_mesh,
    compiler_params=_sc_params,
    scratch_types=[
        pltpu.VMEM((NP,), jnp.float32),
        pltpu.VMEM((NP,), jnp.float32),
        pltpu.VMEM((E // NW,), jnp.int32),
        pltpu.VMEM((E // NW,), jnp.int32),
    ],
)


def _mm1a_body(x_ref, w1_ref, h1t_ref):
    h1 = jnp.dot(x_ref[...], w1_ref[...],
                 preferred_element_type=jnp.float32)
    h1t_ref[...] = jnp.concatenate(
        [h1.T, jnp.zeros((HP, NP - N), jnp.float32)], axis=1)


_mm1a_call = pl.pallas_call(
    _mm1a_body,
    out_shape=jax.ShapeDtypeStruct((HP, NP), jnp.float32),
)


def _mm1b_body(degp_ref, h1t_ref, g1t_ref, dinv_ref):
    degp = degp_ref[...].reshape(NW, NP)
    deg = jnp.sum(degp, axis=0, keepdims=True) + 1.0
    dinv = lax.rsqrt(deg)
    g1t_ref[...] = (h1t_ref[...] * dinv).reshape(HP * NP)
    dinv_ref[...] = dinv


_mm1b_call = pl.pallas_call(
    _mm1b_body,
    out_shape=[
        jax.ShapeDtypeStruct((HP * NP,), jnp.float32),
        jax.ShapeDtypeStruct((1, NP), jnp.float32),
    ],
)


def _mm2_body(p1p_ref, h1t_ref, dinv_ref, b1_ref, w2t_ref, h2_ref, g2_ref):
    p1p = p1p_ref[...].reshape(NS, H, NP)
    p1 = jnp.sum(p1p, axis=0)
    dinv = dinv_ref[...]
    h1t = h1t_ref[...][:H, :]
    out1 = p1 * dinv + (dinv * dinv) * h1t + b1_ref[...]
    out1 = jnp.maximum(out1, 0.0)
    h2 = jnp.dot(w2t_ref[...], out1,
                 preferred_element_type=jnp.float32)
    h2_ref[...] = h2
    g2_ref[...] = (h2 * dinv).reshape(NP)


_mm2_call = pl.pallas_call(
    _mm2_body,
    out_shape=[
        jax.ShapeDtypeStruct((1, NP), jnp.float32),
        jax.ShapeDtypeStruct((NP,), jnp.float32),
    ],
)


def _fin_body(p2p_ref, h2_ref, dinv_ref, b2_ref, out_ref):
    p2p = p2p_ref[...].reshape(NW, NP)
    dinv = dinv_ref[...]
    col = (dinv * jnp.sum(p2p, axis=0, keepdims=True)
           + (dinv * dinv) * h2_ref[...] + b2_ref[...])
    e0 = (lax.broadcasted_iota(jnp.int32, (1, D), 1) == 0)
    out_ref[...] = lax.dot_general(
        col[:, :N], e0.astype(jnp.float32), (((0,), (0,)), ((), ())),
        preferred_element_type=jnp.float32)


_fin_call = pl.pallas_call(
    _fin_body,
    out_shape=jax.ShapeDtypeStruct((N, D), jnp.float32),
)


def kernel(x, edge_index, W1, b1, W2, b2):
    ei = edge_index.astype(jnp.int32).reshape(2 * E)
    w1p = jnp.pad(W1.astype(jnp.float32), ((0, 0), (0, HP - H)))
    b1c = b1.astype(jnp.float32).reshape(H, 1)
    w2t = W2.astype(jnp.float32).reshape(1, H)
    b2c = b2.astype(jnp.float32).reshape(1, 1)
    z_np = jnp.zeros((NP,), jnp.float32)

    degp = _deg_call(ei, z_np)
    h1t = _mm1a_call(x, w1p)
    g1t, dinv = _mm1b_call(degp, h1t)
    p1p = _edge1_call(g1t, ei, z_np)
    h2, g2 = _mm2_call(p1p, h1t, dinv, b1c, w2t)
    p2p = _edge2_call(g2, ei, z_np)
    out = _fin_call(p2p, h2, dinv, b2c)
    return out
